# baseline (speedup 1.0000x reference)
import functools

import jax
import jax.numpy as jnp
from jax import lax
from jax.experimental import pallas as pl
from jax.experimental.pallas import tpu as pltpu
from jax.experimental.pallas import tpu_sc as plsc

N = 10000
E = 160000
D = 256
L = 3
C = 4
G = 64

NP = 10240
NT = 32
AR = NP // NT
CH = 64
NBK = 4 * NT
CPAIR = 2 * CH
E_PAD = E + NBK * CPAIR


_GDN = lax.GatherDimensionNumbers(
    offset_dims=(), collapsed_slice_dims=(0,), start_index_map=(0,))


def _bcast_lane(vec, lane):
    idx = jnp.full((16, 1), lane, jnp.int32)
    return lax.gather(vec, idx, _GDN, (1,),
                      mode=lax.GatherScatterMode.PROMISE_IN_BOUNDS)


def _extract_lane(vec, lane):
    return jnp.sum(jnp.where(lax.iota(jnp.int32, 16) == lane, vec, 0))


@functools.lru_cache(maxsize=None)
def _make_sc_propagate(clusters):
    mesh = plsc.VectorSubcoreMesh(core_axis_name="c", subcore_axis_name="s")

    @functools.partial(
        pl.kernel,
        out_type=jax.ShapeDtypeStruct((NP * D,), jnp.float32),
        mesh=mesh,
        scratch_types=[
            pltpu.VMEM((2 * NBK,), jnp.int32),
            pltpu.VMEM((CH,), jnp.int32),
            pltpu.VMEM((CH,), jnp.int32),
            pltpu.VMEM((CH,), jnp.int32),
            pltpu.VMEM((CH, D), jnp.float32),
            pltpu.VMEM((CH, D), jnp.float32),
            pltpu.VMEM((AR * D,), jnp.float32),
            pltpu.SemaphoreType.DMA,
            pltpu.SemaphoreType.DMA,
        ],
        compiler_params=pltpu.CompilerParams(needs_layout_passes=False),
    )
    def _sc_propagate(params, srcp, didxp, x_hbm, zeros_hbm, agg_hbm,
                      pv, sia, sib, dv, rowa, rowb, acc, sema, semb):
        c = lax.axis_index("c")
        s = lax.axis_index("s")
        w = c * 16 + s
        pltpu.sync_copy(params, pv)
        pltpu.sync_copy(zeros_hbm, acc)

        cols = [lax.iota(jnp.int32, 16) + 16 * j for j in range(16)]

        def compute(chunk_idx, rows_v):
            pltpu.sync_copy(didxp.at[pl.ds(chunk_idx * CH, CH)], dv)

            def grp_body(g, carry2):
                dlv = dv[pl.ds(g * 16, 16)]
                for i in range(16):
                    bl = _bcast_lane(dlv, jnp.int32(i)) * D
                    for j in range(16):
                        v = rows_v[g * 16 + i, pl.ds(16 * j, 16)]
                        plsc.addupdate_scatter(acc, [bl + cols[j]], v)
                return carry2

            lax.fori_loop(0, CH // 16, grp_body, 0)

        def issue(chunk_idx, si, rows_v, sem):
            pltpu.sync_copy(srcp.at[pl.ds(chunk_idx * CH, CH)], si)
            pltpu.async_copy(x_hbm.at[si], rows_v, sem)

        for cl in clusters:
            cs = _extract_lane(pv[pl.ds(cl * 32 + c * 16, 16)], s)
            nch = _extract_lane(pv[pl.ds(NBK + cl * 32 + c * 16, 16)], s)
            issue(cs, sia, rowa, sema)

            def pair_body(jp, carry):
                e0 = cs + 2 * jp
                issue(e0 + 1, sib, rowb, semb)
                pltpu.make_async_copy(x_hbm.at[sia], rowa, sema).wait()
                compute(e0, rowa)
                issue(jnp.minimum(e0 + 2, cs + nch - 2), sia, rowa, sema)
                pltpu.make_async_copy(x_hbm.at[sib], rowb, semb).wait()
                compute(e0 + 1, rowb)
                return carry

            lax.fori_loop(0, nch // 2, pair_body, 0)
            pltpu.make_async_copy(x_hbm.at[sia], rowa, sema).wait()

        pltpu.sync_copy(acc, agg_hbm.at[pl.ds(w * AR * D, AR * D)])

    return _sc_propagate


BLK = 1024
NB = NP // BLK


def _k1_body(x_ref, agg_ref, part_ref, w1_ref, b1_ref,
             h1_ref, stats_ref, aggn_ref, acc):
    pid = pl.program_id(0)
    aggn = agg_ref[...] + part_ref[...]
    aggn_ref[...] = aggn
    out = x_ref[...] + aggn
    h1 = jnp.dot(out.astype(jnp.bfloat16), w1_ref[...].astype(jnp.bfloat16), preferred_element_type=jnp.float32) + b1_ref[...]
    gid = pid * BLK + lax.broadcasted_iota(jnp.int32, (BLK, 1), 0)
    rm = (gid < N).astype(jnp.float32)
    h1m = h1 * rm
    h1_ref[...] = h1

    @pl.when(pid == 0)
    def _():
        acc[...] = jnp.zeros_like(acc)

    acc[0:1, :] += jnp.sum(h1m, axis=0, keepdims=True)

    @pl.when(pid == NB - 1)
    def _():
        stats_ref[...] = acc[...]


_k1 = pl.pallas_call(
    _k1_body,
    grid=(NB,),
    in_specs=[
        pl.BlockSpec((BLK, D), lambda i: (i, 0)),
        pl.BlockSpec((BLK, D), lambda i: (i, 0)),
        pl.BlockSpec((BLK, D), lambda i: (i, 0)),
        pl.BlockSpec((D, D), lambda i: (0, 0)),
        pl.BlockSpec((1, D), lambda i: (0, 0)),
    ],
    out_specs=[
        pl.BlockSpec((BLK, D), lambda i: (i, 0)),
        pl.BlockSpec((8, D), lambda i: (0, 0)),
        pl.BlockSpec((BLK, D), lambda i: (i, 0)),
    ],
    out_shape=[
        jax.ShapeDtypeStruct((NP, D), jnp.float32),
        jax.ShapeDtypeStruct((8, D), jnp.float32),
        jax.ShapeDtypeStruct((NP, D), jnp.float32),
    ],
    scratch_shapes=[pltpu.VMEM((8, D), jnp.float32)],
)


def _var_body(h1_ref, sum_ref, var_ref, acc):
    pid = pl.program_id(0)
    mu = sum_ref[0:1, :] * (1.0 / N)
    gid = pid * BLK + lax.broadcasted_iota(jnp.int32, (BLK, 1), 0)
    rm = (gid < N).astype(jnp.float32)
    d = (h1_ref[...] - mu) * rm

    @pl.when(pid == 0)
    def _():
        acc[...] = jnp.zeros_like(acc)

    acc[0:1, :] += jnp.sum(d * d, axis=0, keepdims=True)

    @pl.when(pid == NB - 1)
    def _():
        var_ref[...] = acc[...]


_kvar = pl.pallas_call(
    _var_body,
    grid=(NB,),
    in_specs=[
        pl.BlockSpec((BLK, D), lambda i: (i, 0)),
        pl.BlockSpec((8, D), lambda i: (0, 0)),
    ],
    out_specs=pl.BlockSpec((8, D), lambda i: (0, 0)),
    out_shape=jax.ShapeDtypeStruct((8, D), jnp.float32),
    scratch_shapes=[pltpu.VMEM((8, D), jnp.float32)],
)


def _bn_apply(h1, sums, vars_, g_ref, be_ref):
    mu = sums[0:1, :] * (1.0 / N)
    var = vars_[0:1, :] * (1.0 / N)
    hn = g_ref[...] * (h1 - mu) / jnp.sqrt(var + 1e-5) + be_ref[...]
    return jnp.maximum(hn, 0.0)


def _mk_k2(cluster, with_pool):
    def body(*refs):
        if with_pool:
            (h1_ref, x_ref, lab_ref, bat_ref, sum_ref, var_ref, w2_ref,
             b2_ref, g_ref, be_ref, xn_ref, delta_ref, pool_ref, pacc) = refs
        else:
            (h1_ref, x_ref, lab_ref, sum_ref, var_ref, w2_ref, b2_ref,
             g_ref, be_ref, xn_ref, delta_ref) = refs
        pid = pl.program_id(0)
        hn = _bn_apply(h1_ref[...], sum_ref[...], var_ref[...], g_ref, be_ref)
        h2 = jnp.dot(hn.astype(jnp.bfloat16), w2_ref[...].astype(jnp.bfloat16), preferred_element_type=jnp.float32) + b2_ref[...]
        xn = jnp.where(lab_ref[...] == cluster, h2, x_ref[...])
        xn_ref[...] = xn
        gid = pid * BLK + lax.broadcasted_iota(jnp.int32, (BLK, 1), 0)
        rm = (gid < N).astype(jnp.float32)
        delta_ref[...] = (h2 - x_ref[...]) * rm
        if with_pool:
            @pl.when(pid == 0)
            def _():
                pacc[...] = jnp.zeros_like(pacc)
            bt = bat_ref[...]
            gids = lax.broadcasted_iota(jnp.int32, (G, 1), 0)
            P = (bt == gids).astype(jnp.float32)
            pacc[...] += jnp.dot(P, xn, preferred_element_type=jnp.float32,
                                 precision=lax.Precision.HIGHEST)

            @pl.when(pid == NB - 1)
            def _():
                pool_ref[...] = pacc[...]

    in_specs = [
        pl.BlockSpec((BLK, D), lambda i: (i, 0)),
        pl.BlockSpec((BLK, D), lambda i: (i, 0)),
        pl.BlockSpec((BLK, 1), lambda i: (i, 0)),
    ]
    if with_pool:
        in_specs.append(pl.BlockSpec((1, BLK), lambda i: (0, i)))
    in_specs += [
        pl.BlockSpec((8, D), lambda i: (0, 0)),
        pl.BlockSpec((8, D), lambda i: (0, 0)),
        pl.BlockSpec((D, D), lambda i: (0, 0)),
        pl.BlockSpec((1, D), lambda i: (0, 0)),
        pl.BlockSpec((1, D), lambda i: (0, 0)),
        pl.BlockSpec((1, D), lambda i: (0, 0)),
    ]
    out_specs = [pl.BlockSpec((BLK, D), lambda i: (i, 0)),
                 pl.BlockSpec((BLK, D), lambda i: (i, 0))]
    out_shape = [jax.ShapeDtypeStruct((NP, D), jnp.float32),
                 jax.ShapeDtypeStruct((NP, D), jnp.float32)]
    scratch = []
    if with_pool:
        out_specs.append(pl.BlockSpec((G, D), lambda i: (0, 0)))
        out_shape.append(jax.ShapeDtypeStruct((G, D), jnp.float32))
        scratch.append(pltpu.VMEM((G, D), jnp.float32))
    return pl.pallas_call(
        body, grid=(NB,), in_specs=in_specs, out_specs=out_specs,
        out_shape=out_shape, scratch_shapes=scratch)


def _pool_body(p0_ref, p1_ref, p2_ref, wp1_ref, bp1_ref, gp_ref, bep_ref,
               wp2_ref, bp2_ref, out_ref):
    h = (jnp.dot(p0_ref[...], wp1_ref[0:D, :], preferred_element_type=jnp.float32)
         + jnp.dot(p1_ref[...], wp1_ref[D:2 * D, :], preferred_element_type=jnp.float32)
         + jnp.dot(p2_ref[...], wp1_ref[2 * D:3 * D, :], preferred_element_type=jnp.float32)
         + bp1_ref[...])
    mu = jnp.mean(h, axis=0, keepdims=True)
    var = jnp.mean((h - mu) * (h - mu), axis=0, keepdims=True)
    hn = gp_ref[...] * (h - mu) / jnp.sqrt(var + 1e-5) + bep_ref[...]
    hn = jnp.maximum(hn, 0.0)
    out_ref[...] = jnp.dot(hn, wp2_ref[...], preferred_element_type=jnp.float32) + bp2_ref[...]


_pool_mlp = pl.pallas_call(
    _pool_body,
    out_shape=jax.ShapeDtypeStruct((G, D), jnp.float32),
)


def kernel(x_feat, W1, b1, g1, be1, W2, b2, Wp1, bp1, gp, bep, Wp2, bp2,
           clustering_labels, edge_index, batch):
    src = edge_index[0].astype(jnp.int32)
    dst = edge_index[1].astype(jnp.int32)

    lab = clustering_labels.astype(jnp.int32)
    key = lab[src] * 16384 + dst
    order = jnp.argsort(key)
    keys_s = key[order]
    dsts = dst[order]
    srcs = src[order]
    br = jnp.arange(NBK + 1, dtype=jnp.int32)
    bvals = (br // NT) * 16384 + (br % NT) * AR
    off = jnp.searchsorted(keys_s, bvals).astype(jnp.int32)
    cnt = off[1:] - off[:-1]
    nch = 2 * ((cnt + CPAIR - 1) // CPAIR)
    cs = jnp.concatenate([jnp.zeros((1,), jnp.int32),
                          jnp.cumsum(nch).astype(jnp.int32)])[:NBK]
    pstart = cs * CH
    params = jnp.concatenate([cs, nch]).astype(jnp.int32)

    q = jnp.arange(E_PAD, dtype=jnp.int32)
    bq = jnp.searchsorted(pstart, q, side="right").astype(jnp.int32) - 1
    rel = q - pstart[bq]
    validq = rel < cnt[bq]
    e_src = jnp.clip(off[:-1][bq] + rel, 0, E - 1)
    srcp = jnp.where(validq, srcs[e_src], N + (q % (NP - N))).astype(jnp.int32)
    didxp = jnp.where(validq, dsts[e_src] - (bq % NT) * AR, 0).astype(jnp.int32)

    zeros_acc = jnp.zeros((AR * D,), jnp.float32)

    xp = jnp.zeros((NP, D), jnp.float32).at[0:N, :].set(x_feat)
    labp = jnp.full((NP, 1), -1, jnp.int32).at[0:N, 0].set(
        clustering_labels.astype(jnp.int32))
    batp = jnp.full((1, NP), 255, jnp.int32).at[0, 0:N].set(batch.astype(jnp.int32))

    pooled = []
    x = xp
    agg = _make_sc_propagate((0, 1, 2, 3))(params, srcp, didxp, x,
                                           zeros_acc).reshape(NP, D)
    part = jnp.zeros((NP, D), jnp.float32)
    for t in range(L):
        for c in range(C):
            i = t * C + c
            h1, sums, aggn = _k1(x, agg, part, W1[i], b1[i].reshape(1, D))
            vars_ = _kvar(h1, sums)
            if c == C - 1:
                k2 = _mk_k2(c, True)
                x_new, delta, pool_t = k2(h1, x, labp, batp, sums, vars_,
                                          W2[i], b2[i].reshape(1, D),
                                          g1[i].reshape(1, D),
                                          be1[i].reshape(1, D))
                pooled.append(pool_t)
            else:
                k2 = _mk_k2(c, False)
                x_new, delta = k2(h1, x, labp, sums, vars_, W2[i],
                                  b2[i].reshape(1, D), g1[i].reshape(1, D),
                                  be1[i].reshape(1, D))
            agg = aggn
            if (t, c) != (L - 1, C - 1):
                part = _make_sc_propagate((c,))(params, srcp, didxp, delta,
                                                zeros_acc).reshape(NP, D)
            x = x_new

    return _pool_mlp(pooled[0], pooled[1], pooled[2], Wp1,
                     bp1.reshape(1, D), gp.reshape(1, D), bep.reshape(1, D),
                     Wp2, bp2.reshape(1, D))


# speedup vs baseline: 1.0000x; 1.0000x over previous
import functools

import jax
import jax.numpy as jnp
from jax import lax
from jax.experimental import pallas as pl
from jax.experimental.pallas import tpu as pltpu
from jax.experimental.pallas import tpu_sc as plsc

N = 10000
E = 160000
D = 256
L = 3
C = 4
G = 64

NP = 10240
NT = 32
AR = NP // NT
CH = 64
NBK = 4 * NT
CPAIR = 2 * CH
E_PAD = E + NBK * CPAIR


_GDN = lax.GatherDimensionNumbers(
    offset_dims=(), collapsed_slice_dims=(0,), start_index_map=(0,))


def _bcast_lane(vec, lane):
    idx = jnp.full((16, 1), lane, jnp.int32)
    return lax.gather(vec, idx, _GDN, (1,),
                      mode=lax.GatherScatterMode.PROMISE_IN_BOUNDS)


def _extract_lane(vec, lane):
    return jnp.sum(jnp.where(lax.iota(jnp.int32, 16) == lane, vec, 0))


@functools.lru_cache(maxsize=None)
def _make_sc_propagate(clusters):
    mesh = plsc.VectorSubcoreMesh(core_axis_name="c", subcore_axis_name="s")

    @functools.partial(
        pl.kernel,
        out_type=jax.ShapeDtypeStruct((NP * D,), jnp.float32),
        mesh=mesh,
        scratch_types=[
            pltpu.VMEM((2 * NBK,), jnp.int32),
            pltpu.VMEM((CH,), jnp.int32),
            pltpu.VMEM((CH,), jnp.int32),
            pltpu.VMEM((CH,), jnp.int32),
            pltpu.VMEM((CH, D), jnp.float32),
            pltpu.VMEM((CH, D), jnp.float32),
            pltpu.VMEM((AR * D,), jnp.float32),
            pltpu.SemaphoreType.DMA,
            pltpu.SemaphoreType.DMA,
        ],
        compiler_params=pltpu.CompilerParams(needs_layout_passes=False),
    )
    def _sc_propagate(params, srcp, didxp, x_hbm, zeros_hbm, agg_hbm,
                      pv, sia, sib, dv, rowa, rowb, acc, sema, semb):
        c = lax.axis_index("c")
        s = lax.axis_index("s")
        w = c * 16 + s
        pltpu.sync_copy(params, pv)
        pltpu.sync_copy(zeros_hbm, acc)

        cols = [lax.iota(jnp.int32, 16) + 16 * j for j in range(16)]

        def compute(chunk_idx, rows_v):
            pltpu.sync_copy(didxp.at[pl.ds(chunk_idx * CH, CH)], dv)

            def grp_body(g, carry2):
                dlv = dv[pl.ds(g * 16, 16)]
                for i in range(16):
                    bl = _bcast_lane(dlv, jnp.int32(i)) * D
                    for j in range(16):
                        v = rows_v[g * 16 + i, pl.ds(16 * j, 16)]
                        plsc.addupdate_scatter(acc, [bl + cols[j]], v)
                return carry2

            lax.fori_loop(0, CH // 16, grp_body, 0)

        def issue(chunk_idx, si, rows_v, sem):
            pltpu.sync_copy(srcp.at[pl.ds(chunk_idx * CH, CH)], si)
            pltpu.async_copy(x_hbm.at[si], rows_v, sem)

        for cl in clusters:
            cs = _extract_lane(pv[pl.ds(cl * 32 + c * 16, 16)], s)
            nch = _extract_lane(pv[pl.ds(NBK + cl * 32 + c * 16, 16)], s)
            issue(cs, sia, rowa, sema)

            def pair_body(jp, carry):
                e0 = cs + 2 * jp
                issue(e0 + 1, sib, rowb, semb)
                pltpu.make_async_copy(x_hbm.at[sia], rowa, sema).wait()
                compute(e0, rowa)
                issue(jnp.minimum(e0 + 2, cs + nch - 2), sia, rowa, sema)
                pltpu.make_async_copy(x_hbm.at[sib], rowb, semb).wait()
                compute(e0 + 1, rowb)
                return carry

            lax.fori_loop(0, nch // 2, pair_body, 0)
            pltpu.make_async_copy(x_hbm.at[sia], rowa, sema).wait()

        pltpu.sync_copy(acc, agg_hbm.at[pl.ds(w * AR * D, AR * D)])

    return _sc_propagate


BLK = 1024
NB = NP // BLK


def _k1_body(x_ref, agg_ref, part_ref, w1_ref, b1_ref,
             h1_ref, stats_ref, aggn_ref, acc):
    pid = pl.program_id(0)
    aggn = agg_ref[...] + part_ref[...]
    aggn_ref[...] = aggn
    out = x_ref[...] + aggn
    h1 = jnp.dot(out, w1_ref[...], preferred_element_type=jnp.float32) + b1_ref[...]
    gid = pid * BLK + lax.broadcasted_iota(jnp.int32, (BLK, 1), 0)
    rm = (gid < N).astype(jnp.float32)
    h1m = h1 * rm
    h1_ref[...] = h1

    @pl.when(pid == 0)
    def _():
        acc[...] = jnp.zeros_like(acc)

    acc[0:1, :] += jnp.sum(h1m, axis=0, keepdims=True)

    @pl.when(pid == NB - 1)
    def _():
        stats_ref[...] = acc[...]


_k1 = pl.pallas_call(
    _k1_body,
    grid=(NB,),
    in_specs=[
        pl.BlockSpec((BLK, D), lambda i: (i, 0)),
        pl.BlockSpec((BLK, D), lambda i: (i, 0)),
        pl.BlockSpec((BLK, D), lambda i: (i, 0)),
        pl.BlockSpec((D, D), lambda i: (0, 0)),
        pl.BlockSpec((1, D), lambda i: (0, 0)),
    ],
    out_specs=[
        pl.BlockSpec((BLK, D), lambda i: (i, 0)),
        pl.BlockSpec((8, D), lambda i: (0, 0)),
        pl.BlockSpec((BLK, D), lambda i: (i, 0)),
    ],
    out_shape=[
        jax.ShapeDtypeStruct((NP, D), jnp.float32),
        jax.ShapeDtypeStruct((8, D), jnp.float32),
        jax.ShapeDtypeStruct((NP, D), jnp.float32),
    ],
    scratch_shapes=[pltpu.VMEM((8, D), jnp.float32)],
)


def _var_body(h1_ref, sum_ref, var_ref, acc):
    pid = pl.program_id(0)
    mu = sum_ref[0:1, :] * (1.0 / N)
    gid = pid * BLK + lax.broadcasted_iota(jnp.int32, (BLK, 1), 0)
    rm = (gid < N).astype(jnp.float32)
    d = (h1_ref[...] - mu) * rm

    @pl.when(pid == 0)
    def _():
        acc[...] = jnp.zeros_like(acc)

    acc[0:1, :] += jnp.sum(d * d, axis=0, keepdims=True)

    @pl.when(pid == NB - 1)
    def _():
        var_ref[...] = acc[...]


_kvar = pl.pallas_call(
    _var_body,
    grid=(NB,),
    in_specs=[
        pl.BlockSpec((BLK, D), lambda i: (i, 0)),
        pl.BlockSpec((8, D), lambda i: (0, 0)),
    ],
    out_specs=pl.BlockSpec((8, D), lambda i: (0, 0)),
    out_shape=jax.ShapeDtypeStruct((8, D), jnp.float32),
    scratch_shapes=[pltpu.VMEM((8, D), jnp.float32)],
)


def _bn_apply(h1, sums, vars_, g_ref, be_ref):
    mu = sums[0:1, :] * (1.0 / N)
    var = vars_[0:1, :] * (1.0 / N)
    hn = g_ref[...] * (h1 - mu) / jnp.sqrt(var + 1e-5) + be_ref[...]
    return jnp.maximum(hn, 0.0)


def _mk_k2(cluster, with_pool):
    def body(*refs):
        if with_pool:
            (h1_ref, x_ref, lab_ref, bat_ref, sum_ref, var_ref, w2_ref,
             b2_ref, g_ref, be_ref, xn_ref, delta_ref, pool_ref, pacc) = refs
        else:
            (h1_ref, x_ref, lab_ref, sum_ref, var_ref, w2_ref, b2_ref,
             g_ref, be_ref, xn_ref, delta_ref) = refs
        pid = pl.program_id(0)
        hn = _bn_apply(h1_ref[...], sum_ref[...], var_ref[...], g_ref, be_ref)
        h2 = jnp.dot(hn, w2_ref[...], preferred_element_type=jnp.float32) + b2_ref[...]
        xn = jnp.where(lab_ref[...] == cluster, h2, x_ref[...])
        xn_ref[...] = xn
        gid = pid * BLK + lax.broadcasted_iota(jnp.int32, (BLK, 1), 0)
        rm = (gid < N).astype(jnp.float32)
        delta_ref[...] = (h2 - x_ref[...]) * rm
        if with_pool:
            @pl.when(pid == 0)
            def _():
                pacc[...] = jnp.zeros_like(pacc)
            bt = bat_ref[...]
            gids = lax.broadcasted_iota(jnp.int32, (G, 1), 0)
            P = (bt == gids).astype(jnp.float32)
            pacc[...] += jnp.dot(P, xn, preferred_element_type=jnp.float32,
                                 precision=lax.Precision.HIGHEST)

            @pl.when(pid == NB - 1)
            def _():
                pool_ref[...] = pacc[...]

    in_specs = [
        pl.BlockSpec((BLK, D), lambda i: (i, 0)),
        pl.BlockSpec((BLK, D), lambda i: (i, 0)),
        pl.BlockSpec((BLK, 1), lambda i: (i, 0)),
    ]
    if with_pool:
        in_specs.append(pl.BlockSpec((1, BLK), lambda i: (0, i)))
    in_specs += [
        pl.BlockSpec((8, D), lambda i: (0, 0)),
        pl.BlockSpec((8, D), lambda i: (0, 0)),
        pl.BlockSpec((D, D), lambda i: (0, 0)),
        pl.BlockSpec((1, D), lambda i: (0, 0)),
        pl.BlockSpec((1, D), lambda i: (0, 0)),
        pl.BlockSpec((1, D), lambda i: (0, 0)),
    ]
    out_specs = [pl.BlockSpec((BLK, D), lambda i: (i, 0)),
                 pl.BlockSpec((BLK, D), lambda i: (i, 0))]
    out_shape = [jax.ShapeDtypeStruct((NP, D), jnp.float32),
                 jax.ShapeDtypeStruct((NP, D), jnp.float32)]
    scratch = []
    if with_pool:
        out_specs.append(pl.BlockSpec((G, D), lambda i: (0, 0)))
        out_shape.append(jax.ShapeDtypeStruct((G, D), jnp.float32))
        scratch.append(pltpu.VMEM((G, D), jnp.float32))
    return pl.pallas_call(
        body, grid=(NB,), in_specs=in_specs, out_specs=out_specs,
        out_shape=out_shape, scratch_shapes=scratch)


def _pool_body(p0_ref, p1_ref, p2_ref, wp1_ref, bp1_ref, gp_ref, bep_ref,
               wp2_ref, bp2_ref, out_ref):
    h = (jnp.dot(p0_ref[...], wp1_ref[0:D, :], preferred_element_type=jnp.float32)
         + jnp.dot(p1_ref[...], wp1_ref[D:2 * D, :], preferred_element_type=jnp.float32)
         + jnp.dot(p2_ref[...], wp1_ref[2 * D:3 * D, :], preferred_element_type=jnp.float32)
         + bp1_ref[...])
    mu = jnp.mean(h, axis=0, keepdims=True)
    var = jnp.mean((h - mu) * (h - mu), axis=0, keepdims=True)
    hn = gp_ref[...] * (h - mu) / jnp.sqrt(var + 1e-5) + bep_ref[...]
    hn = jnp.maximum(hn, 0.0)
    out_ref[...] = jnp.dot(hn, wp2_ref[...], preferred_element_type=jnp.float32) + bp2_ref[...]


_pool_mlp = pl.pallas_call(
    _pool_body,
    out_shape=jax.ShapeDtypeStruct((G, D), jnp.float32),
)


def kernel(x_feat, W1, b1, g1, be1, W2, b2, Wp1, bp1, gp, bep, Wp2, bp2,
           clustering_labels, edge_index, batch):
    src = edge_index[0].astype(jnp.int32)
    dst = edge_index[1].astype(jnp.int32)

    lab = clustering_labels.astype(jnp.int32)
    key = lab[src] * 16384 + dst
    order = jnp.argsort(key)
    keys_s = key[order]
    dsts = dst[order]
    srcs = src[order]
    br = jnp.arange(NBK + 1, dtype=jnp.int32)
    bvals = (br // NT) * 16384 + (br % NT) * AR
    off = jnp.searchsorted(keys_s, bvals).astype(jnp.int32)
    cnt = off[1:] - off[:-1]
    nch = 2 * ((cnt + CPAIR - 1) // CPAIR)
    cs = jnp.concatenate([jnp.zeros((1,), jnp.int32),
                          jnp.cumsum(nch).astype(jnp.int32)])[:NBK]
    pstart = cs * CH
    params = jnp.concatenate([cs, nch]).astype(jnp.int32)

    q = jnp.arange(E_PAD, dtype=jnp.int32)
    bq = jnp.searchsorted(pstart, q, side="right").astype(jnp.int32) - 1
    rel = q - pstart[bq]
    validq = rel < cnt[bq]
    e_src = jnp.clip(off[:-1][bq] + rel, 0, E - 1)
    srcp = jnp.where(validq, srcs[e_src], N + (q % (NP - N))).astype(jnp.int32)
    didxp = jnp.where(validq, dsts[e_src] - (bq % NT) * AR, 0).astype(jnp.int32)

    zeros_acc = jnp.zeros((AR * D,), jnp.float32)

    xp = jnp.zeros((NP, D), jnp.float32).at[0:N, :].set(x_feat)
    labp = jnp.full((NP, 1), -1, jnp.int32).at[0:N, 0].set(
        clustering_labels.astype(jnp.int32))
    batp = jnp.full((1, NP), 255, jnp.int32).at[0, 0:N].set(batch.astype(jnp.int32))

    pooled = []
    x = xp
    agg = _make_sc_propagate((0, 1, 2, 3))(params, srcp, didxp, x,
                                           zeros_acc).reshape(NP, D)
    part = jnp.zeros((NP, D), jnp.float32)
    for t in range(L):
        for c in range(C):
            i = t * C + c
            h1, sums, aggn = _k1(x, agg, part, W1[i], b1[i].reshape(1, D))
            vars_ = _kvar(h1, sums)
            if c == C - 1:
                k2 = _mk_k2(c, True)
                x_new, delta, pool_t = k2(h1, x, labp, batp, sums, vars_,
                                          W2[i], b2[i].reshape(1, D),
                                          g1[i].reshape(1, D),
                                          be1[i].reshape(1, D))
                pooled.append(pool_t)
            else:
                k2 = _mk_k2(c, False)
                x_new, delta = k2(h1, x, labp, sums, vars_, W2[i],
                                  b2[i].reshape(1, D), g1[i].reshape(1, D),
                                  be1[i].reshape(1, D))
            agg = aggn
            if (t, c) != (L - 1, C - 1):
                part = _make_sc_propagate((c,))(params, srcp, didxp, delta,
                                                zeros_acc).reshape(NP, D)
            x = x_new

    return _pool_mlp(pooled[0], pooled[1], pooled[2], Wp1,
                     bp1.reshape(1, D), gp.reshape(1, D), bep.reshape(1, D),
                     Wp2, bp2.reshape(1, D))


# speedup vs baseline: 4.0508x; 4.0507x over previous
import functools

import jax
import jax.numpy as jnp
from jax import lax
from jax.experimental import pallas as pl
from jax.experimental.pallas import tpu as pltpu
from jax.experimental.pallas import tpu_sc as plsc

N = 10000
E = 160000
D = 256
L = 3
C = 4
G = 64

NP = 10240
NT = 32
AR = NP // NT
CH = 64
NBK = 4 * NT
CPAIR = 2 * CH
E_PAD = E + NBK * CPAIR


_GDN = lax.GatherDimensionNumbers(
    offset_dims=(), collapsed_slice_dims=(0,), start_index_map=(0,))


def _bcast_lane(vec, lane):
    idx = jnp.full((16, 1), lane, jnp.int32)
    return lax.gather(vec, idx, _GDN, (1,),
                      mode=lax.GatherScatterMode.PROMISE_IN_BOUNDS)


def _extract_lane(vec, lane):
    return jnp.sum(jnp.where(lax.iota(jnp.int32, 16) == lane, vec, 0))


@functools.lru_cache(maxsize=None)
def _make_sc_propagate(clusters):
    mesh = plsc.VectorSubcoreMesh(core_axis_name="c", subcore_axis_name="s")

    @functools.partial(
        pl.kernel,
        out_type=jax.ShapeDtypeStruct((NP * D,), jnp.float32),
        mesh=mesh,
        scratch_types=[
            pltpu.VMEM((2 * NBK,), jnp.int32),
            pltpu.VMEM((CH,), jnp.int32),
            pltpu.VMEM((CH,), jnp.int32),
            pltpu.VMEM((CH,), jnp.int32),
            pltpu.VMEM((CH, D), jnp.float32),
            pltpu.VMEM((CH, D), jnp.float32),
            pltpu.VMEM((AR * D,), jnp.float32),
            pltpu.SemaphoreType.DMA,
            pltpu.SemaphoreType.DMA,
        ],
        compiler_params=pltpu.CompilerParams(needs_layout_passes=False),
    )
    def _sc_propagate(params, srcp, didxp, x_hbm, zeros_hbm, agg_hbm,
                      pv, sia, sib, dv, rowa, rowb, acc, sema, semb):
        c = lax.axis_index("c")
        s = lax.axis_index("s")
        w = c * 16 + s
        pltpu.sync_copy(params, pv)
        pltpu.sync_copy(zeros_hbm, acc)

        cols = [lax.iota(jnp.int32, 16) + 16 * j for j in range(16)]

        def compute(chunk_idx, rows_v):
            pltpu.sync_copy(didxp.at[pl.ds(chunk_idx * CH, CH)], dv)

            def grp_body(g, carry2):
                dlv = dv[pl.ds(g * 16, 16)]
                for i in range(16):
                    bl = _bcast_lane(dlv, jnp.int32(i)) * D
                    for j in range(16):
                        v = rows_v[g * 16 + i, pl.ds(16 * j, 16)]
                        plsc.addupdate_scatter(acc, [bl + cols[j]], v)
                return carry2

            lax.fori_loop(0, CH // 16, grp_body, 0)

        def issue(chunk_idx, si, rows_v, sem):
            pltpu.sync_copy(srcp.at[pl.ds(chunk_idx * CH, CH)], si)
            pltpu.async_copy(x_hbm.at[si], rows_v, sem)

        for cl in clusters:
            cs = _extract_lane(pv[pl.ds(cl * 32 + c * 16, 16)], s)
            nch = _extract_lane(pv[pl.ds(NBK + cl * 32 + c * 16, 16)], s)
            issue(cs, sia, rowa, sema)

            def pair_body(jp, carry):
                e0 = cs + 2 * jp
                issue(e0 + 1, sib, rowb, semb)
                pltpu.make_async_copy(x_hbm.at[sia], rowa, sema).wait()
                compute(e0, rowa)
                issue(jnp.minimum(e0 + 2, cs + nch - 2), sia, rowa, sema)
                pltpu.make_async_copy(x_hbm.at[sib], rowb, semb).wait()
                compute(e0 + 1, rowb)
                return carry

            lax.fori_loop(0, nch // 2, pair_body, 0)
            pltpu.make_async_copy(x_hbm.at[sia], rowa, sema).wait()

        pltpu.sync_copy(acc, agg_hbm.at[pl.ds(w * AR * D, AR * D)])

    return _sc_propagate


BLK = 1024
NB = NP // BLK


def _k1_body(x_ref, agg_ref, part_ref, w1_ref, b1_ref,
             h1_ref, stats_ref, aggn_ref, acc):
    pid = pl.program_id(0)
    aggn = agg_ref[...] + part_ref[...]
    aggn_ref[...] = aggn
    out = x_ref[...] + aggn
    h1 = jnp.dot(out, w1_ref[...], preferred_element_type=jnp.float32) + b1_ref[...]
    gid = pid * BLK + lax.broadcasted_iota(jnp.int32, (BLK, 1), 0)
    rm = (gid < N).astype(jnp.float32)
    h1m = h1 * rm
    h1_ref[...] = h1

    @pl.when(pid == 0)
    def _():
        acc[...] = jnp.zeros_like(acc)

    acc[0:1, :] += jnp.sum(h1m, axis=0, keepdims=True)

    @pl.when(pid == NB - 1)
    def _():
        stats_ref[...] = acc[...]


_k1 = pl.pallas_call(
    _k1_body,
    grid=(NB,),
    in_specs=[
        pl.BlockSpec((BLK, D), lambda i: (i, 0)),
        pl.BlockSpec((BLK, D), lambda i: (i, 0)),
        pl.BlockSpec((BLK, D), lambda i: (i, 0)),
        pl.BlockSpec((D, D), lambda i: (0, 0)),
        pl.BlockSpec((1, D), lambda i: (0, 0)),
    ],
    out_specs=[
        pl.BlockSpec((BLK, D), lambda i: (i, 0)),
        pl.BlockSpec((8, D), lambda i: (0, 0)),
        pl.BlockSpec((BLK, D), lambda i: (i, 0)),
    ],
    out_shape=[
        jax.ShapeDtypeStruct((NP, D), jnp.float32),
        jax.ShapeDtypeStruct((8, D), jnp.float32),
        jax.ShapeDtypeStruct((NP, D), jnp.float32),
    ],
    scratch_shapes=[pltpu.VMEM((8, D), jnp.float32)],
)


def _var_body(h1_ref, sum_ref, var_ref, acc):
    pid = pl.program_id(0)
    mu = sum_ref[0:1, :] * (1.0 / N)
    gid = pid * BLK + lax.broadcasted_iota(jnp.int32, (BLK, 1), 0)
    rm = (gid < N).astype(jnp.float32)
    d = (h1_ref[...] - mu) * rm

    @pl.when(pid == 0)
    def _():
        acc[...] = jnp.zeros_like(acc)

    acc[0:1, :] += jnp.sum(d * d, axis=0, keepdims=True)

    @pl.when(pid == NB - 1)
    def _():
        var_ref[...] = acc[...]


_kvar = pl.pallas_call(
    _var_body,
    grid=(NB,),
    in_specs=[
        pl.BlockSpec((BLK, D), lambda i: (i, 0)),
        pl.BlockSpec((8, D), lambda i: (0, 0)),
    ],
    out_specs=pl.BlockSpec((8, D), lambda i: (0, 0)),
    out_shape=jax.ShapeDtypeStruct((8, D), jnp.float32),
    scratch_shapes=[pltpu.VMEM((8, D), jnp.float32)],
)


def _bn_apply(h1, sums, vars_, g_ref, be_ref):
    mu = sums[0:1, :] * (1.0 / N)
    var = vars_[0:1, :] * (1.0 / N)
    hn = g_ref[...] * (h1 - mu) / jnp.sqrt(var + 1e-5) + be_ref[...]
    return jnp.maximum(hn, 0.0)


def _mk_k2(cluster, with_pool):
    def body(*refs):
        if with_pool:
            (h1_ref, x_ref, lab_ref, bat_ref, sum_ref, var_ref, w2_ref,
             b2_ref, g_ref, be_ref, xn_ref, delta_ref, pool_ref, pacc) = refs
        else:
            (h1_ref, x_ref, lab_ref, sum_ref, var_ref, w2_ref, b2_ref,
             g_ref, be_ref, xn_ref, delta_ref) = refs
        pid = pl.program_id(0)
        hn = _bn_apply(h1_ref[...], sum_ref[...], var_ref[...], g_ref, be_ref)
        h2 = jnp.dot(hn, w2_ref[...], preferred_element_type=jnp.float32) + b2_ref[...]
        xn = jnp.where(lab_ref[...] == cluster, h2, x_ref[...])
        xn_ref[...] = xn
        gid = pid * BLK + lax.broadcasted_iota(jnp.int32, (BLK, 1), 0)
        rm = (gid < N).astype(jnp.float32)
        delta_ref[...] = (h2 - x_ref[...]) * rm
        if with_pool:
            @pl.when(pid == 0)
            def _():
                pacc[...] = jnp.zeros_like(pacc)
            bt = bat_ref[...]
            gids = lax.broadcasted_iota(jnp.int32, (G, 1), 0)
            P = (bt == gids).astype(jnp.float32)
            pacc[...] += jnp.dot(P, xn, preferred_element_type=jnp.float32,
                                 precision=lax.Precision.HIGHEST)

            @pl.when(pid == NB - 1)
            def _():
                pool_ref[...] = pacc[...]

    in_specs = [
        pl.BlockSpec((BLK, D), lambda i: (i, 0)),
        pl.BlockSpec((BLK, D), lambda i: (i, 0)),
        pl.BlockSpec((BLK, 1), lambda i: (i, 0)),
    ]
    if with_pool:
        in_specs.append(pl.BlockSpec((1, BLK), lambda i: (0, i)))
    in_specs += [
        pl.BlockSpec((8, D), lambda i: (0, 0)),
        pl.BlockSpec((8, D), lambda i: (0, 0)),
        pl.BlockSpec((D, D), lambda i: (0, 0)),
        pl.BlockSpec((1, D), lambda i: (0, 0)),
        pl.BlockSpec((1, D), lambda i: (0, 0)),
        pl.BlockSpec((1, D), lambda i: (0, 0)),
    ]
    out_specs = [pl.BlockSpec((BLK, D), lambda i: (i, 0)),
                 pl.BlockSpec((BLK, D), lambda i: (i, 0))]
    out_shape = [jax.ShapeDtypeStruct((NP, D), jnp.float32),
                 jax.ShapeDtypeStruct((NP, D), jnp.float32)]
    scratch = []
    if with_pool:
        out_specs.append(pl.BlockSpec((G, D), lambda i: (0, 0)))
        out_shape.append(jax.ShapeDtypeStruct((G, D), jnp.float32))
        scratch.append(pltpu.VMEM((G, D), jnp.float32))
    return pl.pallas_call(
        body, grid=(NB,), in_specs=in_specs, out_specs=out_specs,
        out_shape=out_shape, scratch_shapes=scratch)


def _pool_body(p0_ref, p1_ref, p2_ref, wp1_ref, bp1_ref, gp_ref, bep_ref,
               wp2_ref, bp2_ref, out_ref):
    h = (jnp.dot(p0_ref[...], wp1_ref[0:D, :], preferred_element_type=jnp.float32)
         + jnp.dot(p1_ref[...], wp1_ref[D:2 * D, :], preferred_element_type=jnp.float32)
         + jnp.dot(p2_ref[...], wp1_ref[2 * D:3 * D, :], preferred_element_type=jnp.float32)
         + bp1_ref[...])
    mu = jnp.mean(h, axis=0, keepdims=True)
    var = jnp.mean((h - mu) * (h - mu), axis=0, keepdims=True)
    hn = gp_ref[...] * (h - mu) / jnp.sqrt(var + 1e-5) + bep_ref[...]
    hn = jnp.maximum(hn, 0.0)
    out_ref[...] = jnp.dot(hn, wp2_ref[...], preferred_element_type=jnp.float32) + bp2_ref[...]


_pool_mlp = pl.pallas_call(
    _pool_body,
    out_shape=jax.ShapeDtypeStruct((G, D), jnp.float32),
)


def kernel(x_feat, W1, b1, g1, be1, W2, b2, Wp1, bp1, gp, bep, Wp2, bp2,
           clustering_labels, edge_index, batch):
    src = edge_index[0].astype(jnp.int32)
    dst = edge_index[1].astype(jnp.int32)

    lab = clustering_labels.astype(jnp.int32)
    key = lab[src] * 16384 + dst
    keys_s, srcs, dsts = lax.sort((key, src, dst), num_keys=1)
    br = jnp.arange(NBK + 1, dtype=jnp.int32)
    bvals = (br // NT) * 16384 + (br % NT) * AR
    off = jnp.searchsorted(keys_s, bvals).astype(jnp.int32)
    cnt = off[1:] - off[:-1]
    nch = 2 * ((cnt + CPAIR - 1) // CPAIR)
    cs = jnp.concatenate([jnp.zeros((1,), jnp.int32),
                          jnp.cumsum(nch).astype(jnp.int32)])[:NBK]
    pstart = cs * CH
    params = jnp.concatenate([cs, nch]).astype(jnp.int32)

    q = jnp.arange(E_PAD, dtype=jnp.int32)
    le = (pstart[:, None] <= q[None, :])
    bq = jnp.sum(le.astype(jnp.int32), axis=0) - 1
    onehot = (le & ~jnp.concatenate(
        [le[1:], jnp.zeros((1, E_PAD), bool)], axis=0)).astype(jnp.float32)
    tables = jnp.stack([pstart.astype(jnp.float32),
                        cnt.astype(jnp.float32),
                        off[:-1].astype(jnp.float32),
                        (br[:NBK] % NT).astype(jnp.float32)])
    tq = jnp.dot(tables, onehot, precision=lax.Precision.HIGHEST)
    pstart_q = tq[0].astype(jnp.int32)
    cnt_q = tq[1].astype(jnp.int32)
    ostart_q = tq[2].astype(jnp.int32)
    wq = tq[3].astype(jnp.int32)
    rel = q - pstart_q
    validq = rel < cnt_q
    e_src = jnp.clip(ostart_q + rel, 0, E - 1)
    srcp = jnp.where(validq, srcs[e_src], N + (q % (NP - N))).astype(jnp.int32)
    didxp = jnp.where(validq, dsts[e_src] - wq * AR, 0).astype(jnp.int32)

    zeros_acc = jnp.zeros((AR * D,), jnp.float32)

    xp = jnp.zeros((NP, D), jnp.float32).at[0:N, :].set(x_feat)
    labp = jnp.full((NP, 1), -1, jnp.int32).at[0:N, 0].set(
        clustering_labels.astype(jnp.int32))
    batp = jnp.full((1, NP), 255, jnp.int32).at[0, 0:N].set(batch.astype(jnp.int32))

    pooled = []
    x = xp
    agg = _make_sc_propagate((0, 1, 2, 3))(params, srcp, didxp, x,
                                           zeros_acc).reshape(NP, D)
    part = jnp.zeros((NP, D), jnp.float32)
    for t in range(L):
        for c in range(C):
            i = t * C + c
            h1, sums, aggn = _k1(x, agg, part, W1[i], b1[i].reshape(1, D))
            vars_ = _kvar(h1, sums)
            if c == C - 1:
                k2 = _mk_k2(c, True)
                x_new, delta, pool_t = k2(h1, x, labp, batp, sums, vars_,
                                          W2[i], b2[i].reshape(1, D),
                                          g1[i].reshape(1, D),
                                          be1[i].reshape(1, D))
                pooled.append(pool_t)
            else:
                k2 = _mk_k2(c, False)
                x_new, delta = k2(h1, x, labp, sums, vars_, W2[i],
                                  b2[i].reshape(1, D), g1[i].reshape(1, D),
                                  be1[i].reshape(1, D))
            agg = aggn
            if (t, c) != (L - 1, C - 1):
                part = _make_sc_propagate((c,))(params, srcp, didxp, delta,
                                                zeros_acc).reshape(NP, D)
            x = x_new

    return _pool_mlp(pooled[0], pooled[1], pooled[2], Wp1,
                     bp1.reshape(1, D), gp.reshape(1, D), bep.reshape(1, D),
                     Wp2, bp2.reshape(1, D))
